# baseline (speedup 1.0000x reference)
import functools

import jax
import jax.numpy as jnp
from jax import lax
from jax.experimental import pallas as pl
from jax.experimental.pallas import tpu as pltpu
from jax.experimental.pallas import tpu_sc as plsc

N_NODES = 10000
D = 128
N_EDGES = 320000
EPS = 1e-5

NC = 2
NS = 16
DH = D // NC
EPW = N_EDGES // NS
K = 80
NCHUNK = EPW // K
W = 25
NWIN = NCHUNK // W
HBM_WIN = 2
SW = HBM_WIN * W
NBUF = 5
N_PAD = 10240
ROWS_PER_SUB = N_PAD // NS
XROWS_PER_SUB = N_NODES // NS
DEG_W = 16

_mesh = plsc.VectorSubcoreMesh(core_axis_name="c", subcore_axis_name="s")


@functools.partial(
    pl.kernel,
    mesh=_mesh,
    compiler_params=pltpu.CompilerParams(use_tc_tiling_on_sc=False),
    out_type=[
        jax.ShapeDtypeStruct((N_PAD, D), jnp.float32),
        jax.ShapeDtypeStruct((N_PAD, NC * DEG_W), jnp.float32),
    ],
    scratch_types=[
        pltpu.VMEM((2, W, K), jnp.int32),
        pltpu.VMEM((2, W, K), jnp.int32),
        pltpu.VMEM((8, K), jnp.int32),
        pltpu.VMEM((K, DEG_W), jnp.float32),
        *[pltpu.VMEM((K, DH), jnp.float32) for _ in range(NBUF)],
        pltpu.VMEM_SHARED((N_NODES, DH), jnp.float32),
        pltpu.VMEM_SHARED((N_PAD, DH), jnp.float32),
        pltpu.VMEM_SHARED((N_PAD, DEG_W), jnp.float32),
        *[pltpu.SemaphoreType.DMA for _ in range(NBUF)],
        *[pltpu.SemaphoreType.DMA for _ in range(4)],
    ],
)
def _sc_aggregate(x2_hbm, ei_hbm, ei2_hbm, sidx_hbm, zacc_hbm, zdeg_hbm,
                  acc_out_hbm, deg_out_hbm,
                  src_v, dst_v, sidx_v, ones_v, *rest):
    bufs = rest[:NBUF]
    x_s = rest[NBUF]
    acc_s = rest[NBUF + 1]
    deg_s = rest[NBUF + 2]
    sems = rest[NBUF + 3:NBUF + 3 + NBUF]
    isems = rest[NBUF + 3 + NBUF:]

    c = lax.axis_index("c")
    s = lax.axis_index("s")

    def idx_copies(w, slot, hbm):
        if hbm:
            src_ref = ei2_hbm.at[c, s, pl.ds(w * W, W)]
        else:
            src_ref = ei_hbm.at[0, s, pl.ds(w * W, W)]
        return (
            pltpu.make_async_copy(src_ref, src_v.at[slot],
                                  isems[0 + 2 * slot]),
            pltpu.make_async_copy(ei_hbm.at[1, s, pl.ds(w * W, W)],
                                  dst_v.at[slot], isems[1 + 2 * slot]),
        )

    def idx_start(w, slot, hbm):
        for cp in idx_copies(w, slot, hbm):
            cp.start()

    def idx_wait(w, slot, hbm):
        for cp in idx_copies(w, slot, hbm):
            cp.wait()

    idx_start(0, 0, True)

    one16 = jnp.ones((16,), jnp.float32)
    for i in range(K):
        ones_v[i, :] = one16

    xrow0 = s * XROWS_PER_SUB
    row0 = s * ROWS_PER_SUB
    pltpu.sync_copy(zacc_hbm.at[pl.ds(row0, ROWS_PER_SUB)],
                    acc_s.at[pl.ds(row0, ROWS_PER_SUB)])
    pltpu.sync_copy(zdeg_hbm.at[pl.ds(row0, ROWS_PER_SUB)],
                    deg_s.at[pl.ds(row0, ROWS_PER_SUB)])

    pltpu.sync_copy(sidx_hbm.at[c, s], sidx_v)

    def stage_copy(r, b):
        return pltpu.make_async_copy(x2_hbm.at[sidx_v.at[r]], bufs[b],
                                     sems[b])

    for r in range(NBUF):
        stage_copy(r, r).start()
    for r in range(8):
        b = r % NBUF
        stage_copy(r, b).wait()
        cnt = K if r < 7 else XROWS_PER_SUB - 7 * K
        pltpu.sync_copy(bufs[b].at[pl.ds(0, cnt)],
                        x_s.at[pl.ds(xrow0 + K * r, cnt)])
        if r + NBUF < 8:
            stage_copy(r + NBUF, b).start()
    plsc.subcore_barrier()

    idx_wait(0, 0, True)

    def gather_copy(slot, row, b, hbm):
        src_ref = (x2_hbm if hbm else x_s).at[src_v.at[slot, row]]
        return pltpu.make_async_copy(src_ref, bufs[b], sems[b])

    def scatter(j, slot, row, b):
        pltpu.sync_copy(bufs[b], acc_s.at[dst_v.at[slot, row]], add=True)
        do_deg = jnp.logical_xor(j < NCHUNK // 2, c == 1)

        @pl.when(do_deg)
        def _():
            pltpu.sync_copy(ones_v, deg_s.at[dst_v.at[slot, row]], add=True)

    for b in range(NBUF):
        gather_copy(0, b, b, True).start()

    def do_window(base, slot, next_w, cur_hbm, next_hbm, last):
        nslot = 1 - slot
        if not last:
            idx_start(next_w, nslot, next_hbm)
        for k in range(W):
            j = base + k
            gather_copy(slot, k, k % NBUF, cur_hbm).wait()
            scatter(j, slot, k, k % NBUF)
            if last:
                if k < W - NBUF:
                    gather_copy(slot, k + NBUF, k % NBUF, cur_hbm).start()
            else:
                if k == W - NBUF:
                    idx_wait(next_w, nslot, next_hbm)
                la, ls, lh = k + NBUF, slot, cur_hbm
                if la >= W:
                    la, ls, lh = la - W, nslot, next_hbm
                gather_copy(ls, la, k % NBUF, lh).start()

    do_window(0 * W, 0, 1, True, True, False)
    do_window(1 * W, 1, 2, True, False, False)

    def pair_body(p, carry):
        w0 = 2 * p + HBM_WIN
        do_window(w0 * W, 0, w0 + 1, False, False, False)
        do_window((w0 + 1) * W, 1, w0 + 2, False, False, False)
        return carry

    lax.fori_loop(0, (NWIN - HBM_WIN) // 2 - 1, pair_body, 0)

    do_window((NWIN - 2) * W, 0, NWIN - 1, False, False, False)
    do_window((NWIN - 1) * W, 1, 0, False, False, True)

    plsc.subcore_barrier()

    pltpu.sync_copy(acc_s.at[pl.ds(row0, ROWS_PER_SUB)],
                    acc_out_hbm.at[pl.ds(row0, ROWS_PER_SUB),
                                   pl.ds(c * DH, DH)])
    pltpu.sync_copy(deg_s.at[pl.ds(row0, ROWS_PER_SUB)],
                    deg_out_hbm.at[pl.ds(row0, ROWS_PER_SUB),
                                   pl.ds(c * DEG_W, DEG_W)])


def _dense_body(p_ref, dp_ref, w_ref, b_ref, g_ref, be_ref, al_ref, o_ref):
    acc = p_ref[:N_NODES]
    deg = dp_ref[:N_NODES, 0:1] + dp_ref[:N_NODES, DEG_W:DEG_W + 1]
    agg = acc / jnp.maximum(deg, 1.0)
    h = jnp.dot(agg, w_ref[...], preferred_element_type=jnp.float32)
    h = h + b_ref[...]
    mean = jnp.mean(h, axis=0, keepdims=True)
    h_c = h - al_ref[...] * mean
    var = jnp.mean(h_c * h_c, axis=0, keepdims=True)
    out = g_ref[...] * (h_c * lax.rsqrt(var + EPS)) + be_ref[...]
    o_ref[...] = jnp.maximum(out, 0.0)


_dense = pl.pallas_call(
    _dense_body,
    out_shape=jax.ShapeDtypeStruct((N_NODES, D), jnp.float32),
)


def kernel(x, edge_index, W_mat, b, gamma, beta, alpha):
    x2 = x.reshape(NC * N_NODES, DH)
    ei = edge_index.astype(jnp.int32).reshape(2, NS, NCHUNK, K)
    ei2 = 2 * ei[0, :, :SW] + jnp.arange(NC, dtype=jnp.int32).reshape(
        NC, 1, 1, 1)
    rows_pad = (jnp.arange(NS, dtype=jnp.int32)[:, None] * XROWS_PER_SUB
                + jnp.minimum(jnp.arange(8 * K, dtype=jnp.int32),
                              XROWS_PER_SUB - 1)[None, :])
    sidx = (2 * rows_pad.reshape(NS, 8, K)[None]
            + jnp.arange(NC, dtype=jnp.int32).reshape(NC, 1, 1, 1))
    zacc = jnp.zeros((N_PAD, DH), jnp.float32)
    zdeg = jnp.zeros((N_PAD, DEG_W), jnp.float32)
    acc_p, deg_p = _sc_aggregate(x2, ei, ei2, sidx, zacc, zdeg)
    return _dense(acc_p, deg_p, W_mat,
                  b.reshape(1, D), gamma.reshape(1, D),
                  beta.reshape(1, D), alpha.reshape(1, D))


# speedup vs baseline: 1.0618x; 1.0618x over previous
import functools

import jax
import jax.numpy as jnp
from jax import lax
from jax.experimental import pallas as pl
from jax.experimental.pallas import tpu as pltpu
from jax.experimental.pallas import tpu_sc as plsc

N_NODES = 10000
D = 128
N_EDGES = 320000
EPS = 1e-5

NC = 2
NS = 16
DH = D // NC
EPW = N_EDGES // NS
K = 80
NCHUNK = EPW // K
W = 25
NWIN = NCHUNK // W
SPLIT = 22
NBUF = 5
N_PAD = 10240
ROWS_PER_SUB = N_PAD // NS
XROWS_PER_SUB = N_NODES // NS
NSTG = 16
SROWS = 40
DEG_W = 8

_mesh = plsc.VectorSubcoreMesh(core_axis_name="c", subcore_axis_name="s")


@functools.partial(
    pl.kernel,
    mesh=_mesh,
    compiler_params=pltpu.CompilerParams(use_tc_tiling_on_sc=False),
    out_type=[
        jax.ShapeDtypeStruct((N_PAD, D), jnp.float32),
        jax.ShapeDtypeStruct((N_PAD, NC * DEG_W), jnp.float32),
    ],
    scratch_types=[
        pltpu.VMEM((2, W, K), jnp.int32),
        pltpu.VMEM((2, W, K), jnp.int32),
        pltpu.VMEM((NSTG, SROWS), jnp.int32),
        pltpu.VMEM((K, DEG_W), jnp.float32),
        *[pltpu.VMEM((K, DH), jnp.float32) for _ in range(NBUF)],
        *[pltpu.VMEM((SROWS, DH), jnp.float32) for _ in range(2)],
        pltpu.VMEM_SHARED((N_NODES, DH), jnp.float32),
        pltpu.VMEM_SHARED((N_PAD, DH), jnp.float32),
        pltpu.VMEM_SHARED((N_PAD, DEG_W), jnp.float32),
        *[pltpu.SemaphoreType.DMA for _ in range(NBUF)],
        *[pltpu.SemaphoreType.DMA for _ in range(2)],
        *[pltpu.SemaphoreType.DMA for _ in range(4)],
    ],
)
def _sc_aggregate(x2_hbm, esrc_hbm, edst_hbm, sidx_hbm, ones_hbm,
                  zacc_hbm, zdeg_hbm,
                  acc_out_hbm, deg_out_hbm,
                  src_v, dst_v, sidx_v, ones_v, *rest):
    bufs = rest[:NBUF]
    sbufs = rest[NBUF:NBUF + 2]
    x_s = rest[NBUF + 2]
    acc_s = rest[NBUF + 3]
    deg_s = rest[NBUF + 4]
    sems = rest[NBUF + 5:NBUF + 5 + NBUF]
    ssems = rest[NBUF + 5 + NBUF:NBUF + 7 + NBUF]
    isems = rest[NBUF + 7 + NBUF:]

    c = lax.axis_index("c")
    s = lax.axis_index("s")

    def idx_copies(w, slot):
        return (
            pltpu.make_async_copy(esrc_hbm.at[c, s, pl.ds(w * W, W)],
                                  src_v.at[slot], isems[0 + 2 * slot]),
            pltpu.make_async_copy(edst_hbm.at[s, pl.ds(w * W, W)],
                                  dst_v.at[slot], isems[1 + 2 * slot]),
        )

    def idx_start(w, slot):
        for cp in idx_copies(w, slot):
            cp.start()

    def idx_wait(w, slot):
        for cp in idx_copies(w, slot):
            cp.wait()

    idx_start(0, 0)

    pltpu.sync_copy(ones_hbm, ones_v)

    xrow0 = s * XROWS_PER_SUB
    row0 = s * ROWS_PER_SUB
    pltpu.sync_copy(zacc_hbm.at[pl.ds(row0, ROWS_PER_SUB)],
                    acc_s.at[pl.ds(row0, ROWS_PER_SUB)])
    pltpu.sync_copy(zdeg_hbm.at[pl.ds(row0, ROWS_PER_SUB)],
                    deg_s.at[pl.ds(row0, ROWS_PER_SUB)])
    pltpu.sync_copy(sidx_hbm.at[c, s], sidx_v)

    idx_wait(0, 0)

    def stage_copy(r, b):
        return pltpu.make_async_copy(x2_hbm.at[sidx_v.at[r]], sbufs[b],
                                     ssems[b])

    def stage_step(r):
        b = r % 2
        stage_copy(r, b).wait()
        cnt = SROWS if r < NSTG - 1 else XROWS_PER_SUB - (NSTG - 1) * SROWS
        pltpu.sync_copy(sbufs[b].at[pl.ds(0, cnt)],
                        x_s.at[pl.ds(xrow0 + SROWS * r, cnt)])
        if r + 2 < NSTG:
            stage_copy(r + 2, b).start()

    def gather_copy(slot, row, b, hbm):
        src_ref = (x2_hbm if hbm else x_s).at[src_v.at[slot, row]]
        return pltpu.make_async_copy(src_ref, bufs[b], sems[b])

    def scatter(j, slot, row, b):
        pltpu.sync_copy(bufs[b], acc_s.at[dst_v.at[slot, row]], add=True)
        do_deg = jnp.logical_xor(j < NCHUNK // 2, c == 1)

        @pl.when(do_deg)
        def _():
            pltpu.sync_copy(ones_v, deg_s.at[dst_v.at[slot, row]], add=True)

    def do_window(base, slot, next_w, split, next_split, last, aux=None):
        nslot = 1 - slot
        if not last:
            idx_start(next_w, nslot)
        for k in range(W):
            j = base + k
            gather_copy(slot, k, k % NBUF, k < split).wait()
            scatter(j, slot, k, k % NBUF)
            if aux is not None:
                aux(k)
            if last:
                if k < W - NBUF:
                    gather_copy(slot, k + NBUF, k % NBUF,
                                k + NBUF < split).start()
            else:
                if k == W - NBUF:
                    idx_wait(next_w, nslot)
                la = k + NBUF
                if la < W:
                    gather_copy(slot, la, k % NBUF, la < split).start()
                else:
                    gather_copy(nslot, la - W, k % NBUF,
                                la - W < next_split).start()

    for b in range(NBUF):
        gather_copy(0, b, b, True).start()
    stage_copy(0, 0).start()
    stage_copy(1, 1).start()

    do_window(0, 0, 1, W, SPLIT, False,
              aux=lambda k: stage_step(k) if k < NSTG else None)
    plsc.subcore_barrier()

    def pair_body(p, carry):
        w0 = 2 * p + 1
        do_window(w0 * W, 1, w0 + 1, SPLIT, SPLIT, False)
        do_window((w0 + 1) * W, 0, w0 + 2, SPLIT, SPLIT, False)
        return carry

    lax.fori_loop(0, (NWIN - 2) // 2, pair_body, 0)

    do_window((NWIN - 1) * W, 1, 0, SPLIT, SPLIT, True)

    plsc.subcore_barrier()

    pltpu.sync_copy(acc_s.at[pl.ds(row0, ROWS_PER_SUB)],
                    acc_out_hbm.at[pl.ds(row0, ROWS_PER_SUB),
                                   pl.ds(c * DH, DH)])
    pltpu.sync_copy(deg_s.at[pl.ds(row0, ROWS_PER_SUB)],
                    deg_out_hbm.at[pl.ds(row0, ROWS_PER_SUB),
                                   pl.ds(c * DEG_W, DEG_W)])


def _dense_body(p_ref, dp_ref, w_ref, b_ref, g_ref, be_ref, al_ref, o_ref):
    acc = p_ref[:N_NODES]
    deg = dp_ref[:N_NODES, 0:1] + dp_ref[:N_NODES, DEG_W:DEG_W + 1]
    agg = acc / jnp.maximum(deg, 1.0)
    h = jnp.dot(agg, w_ref[...], preferred_element_type=jnp.float32)
    h = h + b_ref[...]
    mean = jnp.mean(h, axis=0, keepdims=True)
    h_c = h - al_ref[...] * mean
    var = jnp.mean(h_c * h_c, axis=0, keepdims=True)
    out = g_ref[...] * (h_c * lax.rsqrt(var + EPS)) + be_ref[...]
    o_ref[...] = jnp.maximum(out, 0.0)


_dense = pl.pallas_call(
    _dense_body,
    out_shape=jax.ShapeDtypeStruct((N_NODES, D), jnp.float32),
)


def kernel(x, edge_index, W_mat, b, gamma, beta, alpha):
    x2 = x.reshape(NC * N_NODES, DH)
    ei = edge_index.astype(jnp.int32).reshape(2, NS, NCHUNK, K)
    j_in_win = jnp.arange(NCHUNK, dtype=jnp.int32) % W
    hbm_chunk = ((jnp.arange(NCHUNK) < W) | (j_in_win < SPLIT))[
        None, None, :, None]
    carange = jnp.arange(NC, dtype=jnp.int32).reshape(NC, 1, 1, 1)
    esrc = jnp.where(hbm_chunk, 2 * ei[0][None] + carange, ei[0][None])
    rows_pad = (jnp.arange(NS, dtype=jnp.int32)[:, None] * XROWS_PER_SUB
                + jnp.minimum(jnp.arange(NSTG * SROWS, dtype=jnp.int32),
                              XROWS_PER_SUB - 1)[None, :])
    sidx = 2 * rows_pad.reshape(NS, NSTG, SROWS)[None] + carange
    ones = jnp.ones((K, DEG_W), jnp.float32)
    zacc = jnp.zeros((N_PAD, DH), jnp.float32)
    zdeg = jnp.zeros((N_PAD, DEG_W), jnp.float32)
    acc_p, deg_p = _sc_aggregate(x2, esrc, ei[1], sidx, ones, zacc, zdeg)
    return _dense(acc_p, deg_p, W_mat,
                  b.reshape(1, D), gamma.reshape(1, D),
                  beta.reshape(1, D), alpha.reshape(1, D))


# speedup vs baseline: 1.1526x; 1.0855x over previous
import functools

import jax
import jax.numpy as jnp
from jax import lax
from jax.experimental import pallas as pl
from jax.experimental.pallas import tpu as pltpu
from jax.experimental.pallas import tpu_sc as plsc

N_NODES = 10000
D = 128
N_EDGES = 320000
EPS = 1e-5

NC = 2
NS = 16
DH = D // NC
EPW = N_EDGES // NS
K = 80
NCHUNK = EPW // K
W = 25
NWIN = NCHUNK // W
SPLIT = 25
NBUF = 5
N_PAD = 10240
ROWS_PER_SUB = N_PAD // NS
XROWS_PER_SUB = N_NODES // NS
NSTG = 16
SROWS = 40
DEG_W = 8

_mesh = plsc.VectorSubcoreMesh(core_axis_name="c", subcore_axis_name="s")


@functools.partial(
    pl.kernel,
    mesh=_mesh,
    compiler_params=pltpu.CompilerParams(use_tc_tiling_on_sc=False),
    out_type=[
        jax.ShapeDtypeStruct((N_PAD, D), jnp.float32),
        jax.ShapeDtypeStruct((N_PAD, NC * DEG_W), jnp.float32),
    ],
    scratch_types=[
        pltpu.VMEM((2, W, K), jnp.int32),
        pltpu.VMEM((2, W, K), jnp.int32),
        pltpu.VMEM((NSTG, SROWS), jnp.int32),
        pltpu.VMEM((K, DEG_W), jnp.float32),
        *[pltpu.VMEM((K, DH), jnp.float32) for _ in range(NBUF)],
        *[pltpu.VMEM((SROWS, DH), jnp.float32) for _ in range(2)],
        pltpu.VMEM_SHARED((N_NODES, DH), jnp.float32),
        pltpu.VMEM_SHARED((N_PAD, DH), jnp.float32),
        pltpu.VMEM_SHARED((N_PAD, DEG_W), jnp.float32),
        *[pltpu.SemaphoreType.DMA for _ in range(NBUF)],
        *[pltpu.SemaphoreType.DMA for _ in range(2)],
        *[pltpu.SemaphoreType.DMA for _ in range(4)],
    ],
)
def _sc_aggregate(x2_hbm, esrc_hbm, edst_hbm, sidx_hbm, ones_hbm,
                  zacc_hbm, zdeg_hbm,
                  acc_out_hbm, deg_out_hbm,
                  src_v, dst_v, sidx_v, ones_v, *rest):
    bufs = rest[:NBUF]
    sbufs = rest[NBUF:NBUF + 2]
    x_s = rest[NBUF + 2]
    acc_s = rest[NBUF + 3]
    deg_s = rest[NBUF + 4]
    sems = rest[NBUF + 5:NBUF + 5 + NBUF]
    ssems = rest[NBUF + 5 + NBUF:NBUF + 7 + NBUF]
    isems = rest[NBUF + 7 + NBUF:]

    c = lax.axis_index("c")
    s = lax.axis_index("s")

    def idx_copies(w, slot):
        return (
            pltpu.make_async_copy(esrc_hbm.at[c, s, pl.ds(w * W, W)],
                                  src_v.at[slot], isems[0 + 2 * slot]),
            pltpu.make_async_copy(edst_hbm.at[s, pl.ds(w * W, W)],
                                  dst_v.at[slot], isems[1 + 2 * slot]),
        )

    def idx_start(w, slot):
        for cp in idx_copies(w, slot):
            cp.start()

    def idx_wait(w, slot):
        for cp in idx_copies(w, slot):
            cp.wait()

    idx_start(0, 0)

    pltpu.sync_copy(ones_hbm, ones_v)

    xrow0 = s * XROWS_PER_SUB
    row0 = s * ROWS_PER_SUB
    pltpu.sync_copy(zacc_hbm.at[pl.ds(row0, ROWS_PER_SUB)],
                    acc_s.at[pl.ds(row0, ROWS_PER_SUB)])
    pltpu.sync_copy(zdeg_hbm.at[pl.ds(row0, ROWS_PER_SUB)],
                    deg_s.at[pl.ds(row0, ROWS_PER_SUB)])
    pltpu.sync_copy(sidx_hbm.at[c, s], sidx_v)

    idx_wait(0, 0)

    def stage_copy(r, b):
        return pltpu.make_async_copy(x2_hbm.at[sidx_v.at[r]], sbufs[b],
                                     ssems[b])

    def stage_step(r):
        b = r % 2
        stage_copy(r, b).wait()
        cnt = SROWS if r < NSTG - 1 else XROWS_PER_SUB - (NSTG - 1) * SROWS
        pltpu.sync_copy(sbufs[b].at[pl.ds(0, cnt)],
                        x_s.at[pl.ds(xrow0 + SROWS * r, cnt)])
        if r + 2 < NSTG:
            stage_copy(r + 2, b).start()

    def gather_copy(slot, row, b, hbm):
        src_ref = (x2_hbm if hbm else x_s).at[src_v.at[slot, row]]
        return pltpu.make_async_copy(src_ref, bufs[b], sems[b])

    def scatter(j, slot, row, b):
        pltpu.sync_copy(bufs[b], acc_s.at[dst_v.at[slot, row]], add=True)
        do_deg = jnp.logical_xor(j < NCHUNK // 2, c == 1)

        @pl.when(do_deg)
        def _():
            pltpu.sync_copy(ones_v, deg_s.at[dst_v.at[slot, row]], add=True)

    def do_window(base, slot, next_w, split, next_split, last, aux=None):
        nslot = 1 - slot
        if not last:
            idx_start(next_w, nslot)
        for k in range(W):
            j = base + k
            gather_copy(slot, k, k % NBUF, k < split).wait()
            scatter(j, slot, k, k % NBUF)
            if aux is not None:
                aux(k)
            if last:
                if k < W - NBUF:
                    gather_copy(slot, k + NBUF, k % NBUF,
                                k + NBUF < split).start()
            else:
                if k == W - NBUF:
                    idx_wait(next_w, nslot)
                la = k + NBUF
                if la < W:
                    gather_copy(slot, la, k % NBUF, la < split).start()
                else:
                    gather_copy(nslot, la - W, k % NBUF,
                                la - W < next_split).start()

    for b in range(NBUF):
        gather_copy(0, b, b, True).start()
    stage_copy(0, 0).start()
    stage_copy(1, 1).start()

    do_window(0, 0, 1, W, SPLIT, False,
              aux=lambda k: stage_step(k) if k < NSTG else None)
    plsc.subcore_barrier()

    def pair_body(p, carry):
        w0 = 2 * p + 1
        do_window(w0 * W, 1, w0 + 1, SPLIT, SPLIT, False)
        do_window((w0 + 1) * W, 0, w0 + 2, SPLIT, SPLIT, False)
        return carry

    lax.fori_loop(0, (NWIN - 2) // 2, pair_body, 0)

    do_window((NWIN - 1) * W, 1, 0, SPLIT, SPLIT, True)

    plsc.subcore_barrier()

    pltpu.sync_copy(acc_s.at[pl.ds(row0, ROWS_PER_SUB)],
                    acc_out_hbm.at[pl.ds(row0, ROWS_PER_SUB),
                                   pl.ds(c * DH, DH)])
    pltpu.sync_copy(deg_s.at[pl.ds(row0, ROWS_PER_SUB)],
                    deg_out_hbm.at[pl.ds(row0, ROWS_PER_SUB),
                                   pl.ds(c * DEG_W, DEG_W)])


def _dense_body(p_ref, dp_ref, w_ref, b_ref, g_ref, be_ref, al_ref, o_ref):
    acc = p_ref[:N_NODES]
    deg = dp_ref[:N_NODES, 0:1] + dp_ref[:N_NODES, DEG_W:DEG_W + 1]
    agg = acc / jnp.maximum(deg, 1.0)
    h = jnp.dot(agg, w_ref[...], preferred_element_type=jnp.float32)
    h = h + b_ref[...]
    mean = jnp.mean(h, axis=0, keepdims=True)
    h_c = h - al_ref[...] * mean
    var = jnp.mean(h_c * h_c, axis=0, keepdims=True)
    out = g_ref[...] * (h_c * lax.rsqrt(var + EPS)) + be_ref[...]
    o_ref[...] = jnp.maximum(out, 0.0)


_dense = pl.pallas_call(
    _dense_body,
    out_shape=jax.ShapeDtypeStruct((N_NODES, D), jnp.float32),
)


def kernel(x, edge_index, W_mat, b, gamma, beta, alpha):
    x2 = x.reshape(NC * N_NODES, DH)
    ei = edge_index.astype(jnp.int32).reshape(2, NS, NCHUNK, K)
    j_in_win = jnp.arange(NCHUNK, dtype=jnp.int32) % W
    hbm_chunk = ((jnp.arange(NCHUNK) < W) | (j_in_win < SPLIT))[
        None, None, :, None]
    carange = jnp.arange(NC, dtype=jnp.int32).reshape(NC, 1, 1, 1)
    esrc = jnp.where(hbm_chunk, 2 * ei[0][None] + carange, ei[0][None])
    rows_pad = (jnp.arange(NS, dtype=jnp.int32)[:, None] * XROWS_PER_SUB
                + jnp.minimum(jnp.arange(NSTG * SROWS, dtype=jnp.int32),
                              XROWS_PER_SUB - 1)[None, :])
    sidx = 2 * rows_pad.reshape(NS, NSTG, SROWS)[None] + carange
    ones = jnp.ones((K, DEG_W), jnp.float32)
    zacc = jnp.zeros((N_PAD, DH), jnp.float32)
    zdeg = jnp.zeros((N_PAD, DEG_W), jnp.float32)
    acc_p, deg_p = _sc_aggregate(x2, esrc, ei[1], sidx, ones, zacc, zdeg)
    return _dense(acc_p, deg_p, W_mat,
                  b.reshape(1, D), gamma.reshape(1, D),
                  beta.reshape(1, D), alpha.reshape(1, D))


# speedup vs baseline: 1.2105x; 1.0502x over previous
import functools

import jax
import jax.numpy as jnp
from jax import lax
from jax.experimental import pallas as pl
from jax.experimental.pallas import tpu as pltpu
from jax.experimental.pallas import tpu_sc as plsc

N_NODES = 10000
D = 128
N_EDGES = 320000
EPS = 1e-5

NC = 2
NS = 16
DH = D // NC
EPW = N_EDGES // NS
K = 80
NCHUNK = EPW // K
NBUF = 5
N_PAD = 10240
ROWS_PER_SUB = N_PAD // NS
DEG_W = 16
LPR = K // 16

_mesh = plsc.VectorSubcoreMesh(core_axis_name="c", subcore_axis_name="s")


@functools.partial(
    pl.kernel,
    mesh=_mesh,
    compiler_params=pltpu.CompilerParams(use_tc_tiling_on_sc=False),
    out_type=[
        jax.ShapeDtypeStruct((N_PAD, D), jnp.float32),
        jax.ShapeDtypeStruct((N_PAD, NC * DEG_W), jnp.float32),
    ],
    scratch_types=[
        pltpu.VMEM((NCHUNK, K), jnp.int32),
        pltpu.VMEM((NCHUNK, K), jnp.int32),
        pltpu.VMEM((K, DEG_W), jnp.float32),
        *[pltpu.VMEM((K, DH), jnp.float32) for _ in range(NBUF)],
        pltpu.VMEM_SHARED((N_PAD, DH), jnp.float32),
        pltpu.VMEM_SHARED((N_PAD, DEG_W), jnp.float32),
        *[pltpu.SemaphoreType.DMA for _ in range(NBUF)],
    ],
)
def _sc_aggregate(x2_hbm, esrc_hbm, edst_hbm, zacc_hbm, zdeg_hbm,
                  acc_out_hbm, deg_out_hbm,
                  src_v, dst_v, ones_v, *rest):
    bufs = rest[:NBUF]
    acc_s = rest[NBUF]
    deg_s = rest[NBUF + 1]
    sems = rest[NBUF + 2:NBUF + 2 + NBUF]

    c = lax.axis_index("c")
    s = lax.axis_index("s")

    pltpu.sync_copy(esrc_hbm.at[c, s], src_v)
    pltpu.sync_copy(edst_hbm.at[s], dst_v)

    one16 = jnp.ones((16,), jnp.float32)
    for i in range(K):
        ones_v[i, :] = one16

    row0 = s * ROWS_PER_SUB
    pltpu.sync_copy(zacc_hbm.at[pl.ds(row0, ROWS_PER_SUB)],
                    acc_s.at[pl.ds(row0, ROWS_PER_SUB)])
    pltpu.sync_copy(zdeg_hbm.at[pl.ds(row0, ROWS_PER_SUB)],
                    deg_s.at[pl.ds(row0, ROWS_PER_SUB)])
    plsc.subcore_barrier()

    def gather_start(j, b):
        pltpu.make_async_copy(x2_hbm.at[src_v.at[j]], bufs[b], sems[b]).start()

    def gather_wait(j, b):
        pltpu.make_async_copy(x2_hbm.at[src_v.at[j]], bufs[b], sems[b]).wait()

    def scatter(j, b):
        pltpu.sync_copy(bufs[b], acc_s.at[dst_v.at[j]], add=True)
        do_deg = jnp.logical_xor(j < NCHUNK // 2, c == 1)

        @pl.when(do_deg)
        def _():
            pltpu.sync_copy(ones_v, deg_s.at[dst_v.at[j]], add=True)

    for b in range(NBUF):
        gather_start(b, b)

    def outer(o, carry):
        base = o * NBUF
        for b in range(NBUF):
            j = base + b
            gather_wait(j, b)
            scatter(j, b)
            gather_start(j + NBUF, b)
        return carry

    lax.fori_loop(0, NCHUNK // NBUF - 1, outer, 0)

    base = NCHUNK - NBUF
    for b in range(NBUF):
        gather_wait(base + b, b)
        scatter(base + b, b)

    plsc.subcore_barrier()

    pltpu.sync_copy(acc_s.at[pl.ds(row0, ROWS_PER_SUB)],
                    acc_out_hbm.at[pl.ds(row0, ROWS_PER_SUB),
                                   pl.ds(c * DH, DH)])
    pltpu.sync_copy(deg_s.at[pl.ds(row0, ROWS_PER_SUB)],
                    deg_out_hbm.at[pl.ds(row0, ROWS_PER_SUB),
                                   pl.ds(c * DEG_W, DEG_W)])


def _dense_body(p_ref, dp_ref, w_ref, b_ref, g_ref, be_ref, al_ref, o_ref):
    acc = p_ref[:N_NODES]
    deg = dp_ref[:N_NODES, 0:1] + dp_ref[:N_NODES, DEG_W:DEG_W + 1]
    agg = acc / jnp.maximum(deg, 1.0)
    h = jnp.dot(agg, w_ref[...], preferred_element_type=jnp.float32)
    h = h + b_ref[...]
    mean = jnp.mean(h, axis=0, keepdims=True)
    h_c = h - al_ref[...] * mean
    var = jnp.mean(h_c * h_c, axis=0, keepdims=True)
    out = g_ref[...] * (h_c * lax.rsqrt(var + EPS)) + be_ref[...]
    o_ref[...] = jnp.maximum(out, 0.0)


_dense = pl.pallas_call(
    _dense_body,
    out_shape=jax.ShapeDtypeStruct((N_NODES, D), jnp.float32),
)


def kernel(x, edge_index, W, b, gamma, beta, alpha):
    x2 = x.reshape(NC * N_NODES, DH)
    ei = edge_index.astype(jnp.int32).reshape(2, NS, NCHUNK, K)
    esrc = (2 * ei[0][None]
            + jnp.arange(NC, dtype=jnp.int32).reshape(NC, 1, 1, 1))
    zacc = jnp.zeros((N_PAD, DH), jnp.float32)
    zdeg = jnp.zeros((N_PAD, DEG_W), jnp.float32)
    acc_p, deg_p = _sc_aggregate(x2, esrc, ei[1], zacc, zdeg)
    return _dense(acc_p, deg_p, W,
                  b.reshape(1, D), gamma.reshape(1, D),
                  beta.reshape(1, D), alpha.reshape(1, D))


# speedup vs baseline: 1.3842x; 1.1435x over previous
import functools

import jax
import jax.numpy as jnp
from jax import lax
from jax.experimental import pallas as pl
from jax.experimental.pallas import tpu as pltpu
from jax.experimental.pallas import tpu_sc as plsc

N_NODES = 10000
D = 128
N_EDGES = 320000
EPS = 1e-5

NC = 2
NS = 16
DH = D // NC
EPW = N_EDGES // NS
K = 80
NCHUNK = EPW // K
NBUF = 5
N_PAD = 10240
ROWS_PER_SUB = N_PAD // NS
DEG_W = 16
LPR = K // 16

_mesh = plsc.VectorSubcoreMesh(core_axis_name="c", subcore_axis_name="s")


@functools.partial(
    pl.kernel,
    mesh=_mesh,
    compiler_params=pltpu.CompilerParams(use_tc_tiling_on_sc=False),
    out_type=[
        jax.ShapeDtypeStruct((N_PAD, D), jnp.float32),
        jax.ShapeDtypeStruct((N_PAD, NC * DEG_W), jnp.float32),
    ],
    scratch_types=[
        pltpu.VMEM((NCHUNK, K), jnp.int32),
        pltpu.VMEM((NCHUNK, K), jnp.int32),
        pltpu.VMEM((K, DEG_W), jnp.float32),
        *[pltpu.VMEM((K, DH), jnp.float32) for _ in range(NBUF)],
        pltpu.VMEM_SHARED((N_PAD, DH), jnp.float32),
        pltpu.VMEM_SHARED((N_PAD, DEG_W), jnp.float32),
        *[pltpu.SemaphoreType.DMA for _ in range(NBUF)],
    ],
)
def _sc_aggregate(x2_hbm, ei_hbm, zacc_hbm, zdeg_hbm,
                  acc_out_hbm, deg_out_hbm,
                  src_v, dst_v, ones_v, *rest):
    bufs = rest[:NBUF]
    acc_s = rest[NBUF]
    deg_s = rest[NBUF + 1]
    sems = rest[NBUF + 2:NBUF + 2 + NBUF]

    c = lax.axis_index("c")
    s = lax.axis_index("s")

    pltpu.sync_copy(ei_hbm.at[0, s], src_v)
    pltpu.sync_copy(ei_hbm.at[1, s], dst_v)

    two = jnp.full((16,), 2, jnp.int32)
    coff = jnp.full((16,), 1, jnp.int32) * c

    def adj(i, carry):
        r = i // LPR
        col = (i % LPR) * 16
        v = src_v[r, pl.ds(col, 16)]
        src_v[r, pl.ds(col, 16)] = v * two + coff
        return carry

    lax.fori_loop(0, NCHUNK * LPR, adj, 0)

    one16 = jnp.ones((16,), jnp.float32)
    for i in range(K):
        ones_v[i, :] = one16

    row0 = s * ROWS_PER_SUB
    pltpu.sync_copy(zacc_hbm.at[pl.ds(row0, ROWS_PER_SUB)],
                    acc_s.at[pl.ds(row0, ROWS_PER_SUB)])
    pltpu.sync_copy(zdeg_hbm.at[pl.ds(row0, ROWS_PER_SUB)],
                    deg_s.at[pl.ds(row0, ROWS_PER_SUB)])
    plsc.subcore_barrier()

    def gather_start(j, b):
        pltpu.make_async_copy(x2_hbm.at[src_v.at[j]], bufs[b], sems[b]).start()

    def gather_wait(j, b):
        pltpu.make_async_copy(x2_hbm.at[src_v.at[j]], bufs[b], sems[b]).wait()

    def scatter(j, b):
        pltpu.sync_copy(bufs[b], acc_s.at[dst_v.at[j]], add=True)
        do_deg = jnp.logical_xor(j < NCHUNK // 2, c == 1)

        @pl.when(do_deg)
        def _():
            pltpu.sync_copy(ones_v, deg_s.at[dst_v.at[j]], add=True)

    for b in range(NBUF):
        gather_start(b, b)

    def outer(o, carry):
        base = o * NBUF
        for b in range(NBUF):
            j = base + b
            gather_wait(j, b)
            scatter(j, b)
            gather_start(j + NBUF, b)
        return carry

    lax.fori_loop(0, NCHUNK // NBUF - 1, outer, 0)

    base = NCHUNK - NBUF
    for b in range(NBUF):
        gather_wait(base + b, b)
        scatter(base + b, b)

    plsc.subcore_barrier()

    pltpu.sync_copy(acc_s.at[pl.ds(row0, ROWS_PER_SUB)],
                    acc_out_hbm.at[pl.ds(row0, ROWS_PER_SUB),
                                   pl.ds(c * DH, DH)])
    pltpu.sync_copy(deg_s.at[pl.ds(row0, ROWS_PER_SUB)],
                    deg_out_hbm.at[pl.ds(row0, ROWS_PER_SUB),
                                   pl.ds(c * DEG_W, DEG_W)])


def _dense_body(p_ref, dp_ref, w_ref, b_ref, g_ref, be_ref, al_ref, o_ref):
    acc = p_ref[:N_NODES]
    deg = dp_ref[:N_NODES, 0:1] + dp_ref[:N_NODES, DEG_W:DEG_W + 1]
    agg = acc / jnp.maximum(deg, 1.0)
    h = jnp.dot(agg, w_ref[...], preferred_element_type=jnp.float32)
    h = h + b_ref[...]
    mean = jnp.mean(h, axis=0, keepdims=True)
    h_c = h - al_ref[...] * mean
    var = jnp.mean(h_c * h_c, axis=0, keepdims=True)
    out = g_ref[...] * (h_c * lax.rsqrt(var + EPS)) + be_ref[...]
    o_ref[...] = jnp.maximum(out, 0.0)


_dense = pl.pallas_call(
    _dense_body,
    out_shape=jax.ShapeDtypeStruct((N_NODES, D), jnp.float32),
)


def kernel(x, edge_index, W, b, gamma, beta, alpha):
    x2 = x.reshape(NC * N_NODES, DH)
    ei = edge_index.astype(jnp.int32).reshape(2, NS, NCHUNK, K)
    zacc = jnp.zeros((N_PAD, DH), jnp.float32)
    zdeg = jnp.zeros((N_PAD, DEG_W), jnp.float32)
    acc_p, deg_p = _sc_aggregate(x2, ei, zacc, zdeg)
    return _dense(acc_p, deg_p, W,
                  b.reshape(1, D), gamma.reshape(1, D),
                  beta.reshape(1, D), alpha.reshape(1, D))
